# baseline (speedup 1.0000x reference)
import functools
import math

import jax
import jax.numpy as jnp
from jax import lax
from jax.experimental import pallas as pl
from jax.experimental.pallas import tpu as pltpu
from jax.experimental.pallas import tpu_sc as plsc

EMB = 64
SCALE = math.sqrt(EMB)
NC = 2
NS = 16
NW = NC * NS
LANES = 16
CHUNK = 128
NBUF = 4
LA = 2


def _emb_body(n_chunks, nb, tok_hbm, tab_hbm, out_hbm,
              idx_v, a_v, t_v, sem_i, sem_g, sem_w):
    wid = lax.axis_index("s") * NC + lax.axis_index("c")
    c0 = wid * n_chunks

    def fire_idx(j):
        c = c0 + j
        s, b = c // nb, c % nb
        pltpu.async_copy(tok_hbm.at[s, pl.ds(b * CHUNK, CHUNK)],
                         idx_v.at[j % NBUF], sem_i)

    def fire_gather(j):
        pltpu.async_copy(tab_hbm.at[idx_v.at[j % NBUF]], a_v.at[j % NBUF], sem_g)

    def wait_idx():
        pltpu.make_async_copy(tok_hbm.at[0, pl.ds(0, CHUNK)],
                              idx_v.at[0], sem_i).wait()

    def wait_gather():
        pltpu.make_async_copy(tab_hbm.at[pl.ds(0, CHUNK)],
                              a_v.at[0], sem_g).wait()

    def wait_write():
        for _ in range(EMB // 8):
            pltpu.make_async_copy(t_v.at[0, 0], out_hbm.at[0, 0, 0],
                                  sem_w).wait()

    lanes = lax.iota(jnp.int32, LANES)
    gvecs = [(lanes >> 3) + 2 * c for c in range(EMB // LANES)]
    evecs = [lanes & 7 for _ in range(EMB // LANES)]

    for j in range(min(NBUF, n_chunks)):
        fire_idx(j)
    for j in range(min(LA, n_chunks)):
        wait_idx()
        fire_gather(j)

    def step(j, _):
        slot = j % NBUF
        c = c0 + j
        s, b = c // nb, c % nb

        @pl.when(j >= LA)
        def _():
            wait_write()

        @pl.when(j + LA < n_chunks)
        def _():
            wait_idx()
            fire_gather(j + LA)

        wait_gather()

        def trans_tok(bi, _):
            cols = jnp.full((LANES,), bi, dtype=jnp.int32)
            for c4 in range(EMB // LANES):
                v = a_v[slot, bi, pl.ds(c4 * LANES, LANES)] * SCALE
                plsc.store_scatter(t_v.at[slot], [gvecs[c4], evecs[c4], cols], v)
            return 0

        lax.fori_loop(0, CHUNK, trans_tok, 0, unroll=4)

        for g in range(EMB // 8):
            pltpu.async_copy(t_v.at[slot, g], out_hbm.at[s, g, b], sem_w)

        @pl.when(j + NBUF < n_chunks)
        def _():
            fire_idx(j + NBUF)
        return 0

    lax.fori_loop(0, n_chunks, step, 0)
    for _ in range(min(LA, n_chunks)):
        wait_write()


def kernel(tokens, table):
    bsz, s_len = tokens.shape
    nb = bsz // CHUNK
    total_chunks = s_len * nb
    assert total_chunks % NW == 0
    n_chunks = total_chunks // NW

    tok_t = tokens.T.astype(jnp.int32)
    t2 = lax.optimization_barrier(table.reshape(table.shape[0] // 2, 2 * EMB))
    t3 = t2.reshape(table.shape[0], EMB)

    mesh = plsc.VectorSubcoreMesh(
        core_axis_name="c", subcore_axis_name="s", num_cores=NC, num_subcores=NS
    )
    emb = pl.kernel(
        functools.partial(_emb_body, n_chunks, nb),
        out_type=jax.ShapeDtypeStruct((s_len, EMB // 8, nb, 8, CHUNK), jnp.float32),
        mesh=mesh,
        scratch_types=[
            pltpu.VMEM((NBUF, CHUNK), jnp.int32),
            pltpu.VMEM((NBUF, CHUNK, EMB), jnp.float32),
            pltpu.VMEM((NBUF, EMB // 8, 8, CHUNK), jnp.float32),
            pltpu.SemaphoreType.DMA,
            pltpu.SemaphoreType.DMA,
            pltpu.SemaphoreType.DMA,
        ],
        compiler_params=pltpu.CompilerParams(
            use_tc_tiling_on_sc=False, needs_layout_passes=False
        ),
    )
    out5 = emb(tok_t, t3)
    return out5.transpose(2, 4, 0, 1, 3).reshape(bsz, s_len, EMB)


# speedup vs baseline: 1.0010x; 1.0010x over previous
import functools
import math

import jax
import jax.numpy as jnp
from jax import lax
from jax.experimental import pallas as pl
from jax.experimental.pallas import tpu as pltpu
from jax.experimental.pallas import tpu_sc as plsc

EMB = 64
SCALE = math.sqrt(EMB)
NC = 2
NS = 16
NW = NC * NS
LANES = 16
CHUNK = 128
TILE = 1024
NBUF = 4
LA = 2


def _emb_body(n_chunks, nb, tok_hbm, tab_hbm, out_hbm,
              idx_v, a_v, t_v, sem_i, sem_g, sem_w):
    wid = lax.axis_index("s") * NC + lax.axis_index("c")
    c0 = wid * n_chunks

    def fire_idx(j):
        c = c0 + j
        s, b = c // nb, c % nb
        pltpu.async_copy(tok_hbm.at[s, pl.ds(b * CHUNK, CHUNK)],
                         idx_v.at[j % NBUF], sem_i)

    def fire_gather(j):
        pltpu.async_copy(tab_hbm.at[idx_v.at[j % NBUF]], a_v.at[j % NBUF], sem_g)

    def wait_idx():
        pltpu.make_async_copy(tok_hbm.at[0, pl.ds(0, CHUNK)],
                              idx_v.at[0], sem_i).wait()

    def wait_gather():
        pltpu.make_async_copy(tab_hbm.at[pl.ds(0, CHUNK)],
                              a_v.at[0], sem_g).wait()

    def wait_write():
        for _ in range(EMB // 8):
            pltpu.make_async_copy(t_v.at[0, pl.ds(0, TILE)],
                                  out_hbm.at[0, 0, 0], sem_w).wait()

    lanes = lax.iota(jnp.int32, LANES)
    rowflat = [(lanes + c * LANES) * CHUNK for c in range(EMB // LANES)]

    for j in range(min(NBUF, n_chunks)):
        fire_idx(j)
    for j in range(min(LA, n_chunks)):
        wait_idx()
        fire_gather(j)

    def step(j, _):
        slot = j % NBUF
        c = c0 + j
        s, b = c // nb, c % nb

        @pl.when(j >= LA)
        def _():
            wait_write()

        @pl.when(j + LA < n_chunks)
        def _():
            wait_idx()
            fire_gather(j + LA)

        wait_gather()

        def trans_tok(bi, _):
            cols = jnp.full((LANES,), bi, dtype=jnp.int32)
            for c4 in range(EMB // LANES):
                v = a_v[slot, bi, pl.ds(c4 * LANES, LANES)] * SCALE
                plsc.store_scatter(t_v.at[slot], [rowflat[c4] + cols], v)
            return 0

        lax.fori_loop(0, CHUNK, trans_tok, 0, unroll=8)

        for g in range(EMB // 8):
            pltpu.async_copy(t_v.at[slot, pl.ds(g * TILE, TILE)],
                             out_hbm.at[s, g, b], sem_w)

        @pl.when(j + NBUF < n_chunks)
        def _():
            fire_idx(j + NBUF)
        return 0

    lax.fori_loop(0, n_chunks, step, 0)
    for _ in range(min(LA, n_chunks)):
        wait_write()


def kernel(tokens, table):
    bsz, s_len = tokens.shape
    nb = bsz // CHUNK
    total_chunks = s_len * nb
    assert total_chunks % NW == 0
    n_chunks = total_chunks // NW

    tok_t = tokens.T.astype(jnp.int32)
    t3 = lax.optimization_barrier(table.T).T

    mesh = plsc.VectorSubcoreMesh(
        core_axis_name="c", subcore_axis_name="s", num_cores=NC, num_subcores=NS
    )
    emb = pl.kernel(
        functools.partial(_emb_body, n_chunks, nb),
        out_type=jax.ShapeDtypeStruct((s_len, EMB // 8, nb, 8 * CHUNK), jnp.float32),
        mesh=mesh,
        scratch_types=[
            pltpu.VMEM((NBUF, CHUNK), jnp.int32),
            pltpu.VMEM((NBUF, CHUNK, EMB), jnp.float32),
            pltpu.VMEM((NBUF, EMB * CHUNK), jnp.float32),
            pltpu.SemaphoreType.DMA,
            pltpu.SemaphoreType.DMA,
            pltpu.SemaphoreType.DMA,
        ],
        compiler_params=pltpu.CompilerParams(
            use_tc_tiling_on_sc=False,
            needs_layout_passes=False,
            disable_bounds_checks=True,
        ),
    )
    out4 = emb(tok_t, t3)
    out5 = out4.reshape(s_len, EMB // 8, nb, 8, CHUNK)
    return out5.transpose(2, 4, 0, 1, 3).reshape(bsz, s_len, EMB)


# speedup vs baseline: 1.3078x; 1.3065x over previous
import functools
import math

import jax
import jax.numpy as jnp
from jax import lax
from jax.experimental import pallas as pl
from jax.experimental.pallas import tpu as pltpu
from jax.experimental.pallas import tpu_sc as plsc

EMB = 64
SCALE = math.sqrt(EMB)
NC = 2
NS = 16
NW = NC * NS
LANES = 16
CHUNK = 128
TILE = 1024
NBUF = 4
LA = 2


def _emb_body(n_chunks, nb, tok_hbm, tab_hbm, out_hbm,
              idx_v, a_v, t_v, sem_i, sem_g, sem_w):
    wid = lax.axis_index("s") * NC + lax.axis_index("c")
    c0 = wid * n_chunks

    def fire_idx(j):
        c = c0 + j
        s, b = c // nb, c % nb
        pltpu.async_copy(tok_hbm.at[s, pl.ds(b * CHUNK, CHUNK)],
                         idx_v.at[j % NBUF], sem_i)

    def fire_gather(j):
        pltpu.async_copy(tab_hbm.at[idx_v.at[j % NBUF]], a_v.at[j % NBUF], sem_g)

    def wait_idx():
        pltpu.make_async_copy(tok_hbm.at[0, pl.ds(0, CHUNK)],
                              idx_v.at[0], sem_i).wait()

    def wait_gather():
        pltpu.make_async_copy(tab_hbm.at[pl.ds(0, CHUNK)],
                              a_v.at[0], sem_g).wait()

    def wait_write():
        for _ in range(EMB // 8):
            pltpu.make_async_copy(t_v.at[0, pl.ds(0, TILE)],
                                  out_hbm.at[0, 0, 0], sem_w).wait()

    lanes = lax.iota(jnp.int32, LANES)
    rowflat = [(lanes + c * LANES) * CHUNK for c in range(EMB // LANES)]

    for j in range(min(NBUF, n_chunks)):
        fire_idx(j)
    for j in range(min(LA, n_chunks)):
        wait_idx()
        fire_gather(j)

    def step(j, _):
        slot = j % NBUF
        c = c0 + j
        s, b = c // nb, c % nb

        @pl.when(j >= LA)
        def _():
            wait_write()

        @pl.when(j + LA < n_chunks)
        def _():
            wait_idx()
            fire_gather(j + LA)

        wait_gather()

        @plsc.parallel_loop(0, CHUNK, unroll=8)
        def _(bi):
            cols = jnp.full((LANES,), bi, dtype=jnp.int32)
            for c4 in range(EMB // LANES):
                v = a_v[slot, bi, pl.ds(c4 * LANES, LANES)] * SCALE
                plsc.store_scatter(t_v.at[slot], [rowflat[c4] + cols], v)

        for g in range(EMB // 8):
            pltpu.async_copy(t_v.at[slot, pl.ds(g * TILE, TILE)],
                             out_hbm.at[s, g, b], sem_w)

        @pl.when(j + NBUF < n_chunks)
        def _():
            fire_idx(j + NBUF)
        return 0

    lax.fori_loop(0, n_chunks, step, 0)
    for _ in range(min(LA, n_chunks)):
        wait_write()


def kernel(tokens, table):
    bsz, s_len = tokens.shape
    nb = bsz // CHUNK
    total_chunks = s_len * nb
    assert total_chunks % NW == 0
    n_chunks = total_chunks // NW

    tok_t = tokens.T.astype(jnp.int32)
    t3 = lax.optimization_barrier(table.T).T

    mesh = plsc.VectorSubcoreMesh(
        core_axis_name="c", subcore_axis_name="s", num_cores=NC, num_subcores=NS
    )
    emb = pl.kernel(
        functools.partial(_emb_body, n_chunks, nb),
        out_type=jax.ShapeDtypeStruct((s_len, EMB // 8, nb, 8 * CHUNK), jnp.float32),
        mesh=mesh,
        scratch_types=[
            pltpu.VMEM((NBUF, CHUNK), jnp.int32),
            pltpu.VMEM((NBUF, CHUNK, EMB), jnp.float32),
            pltpu.VMEM((NBUF, EMB * CHUNK), jnp.float32),
            pltpu.SemaphoreType.DMA,
            pltpu.SemaphoreType.DMA,
            pltpu.SemaphoreType.DMA,
        ],
        compiler_params=pltpu.CompilerParams(
            use_tc_tiling_on_sc=False,
            needs_layout_passes=False,
            disable_bounds_checks=True,
        ),
    )
    out4 = emb(tok_t, t3)
    out5 = out4.reshape(s_len, EMB // 8, nb, 8, CHUNK)
    return out5.transpose(2, 4, 0, 1, 3).reshape(bsz, s_len, EMB)


# speedup vs baseline: 2.2622x; 1.7298x over previous
import functools
import math

import jax
import jax.numpy as jnp
from jax import lax
from jax.experimental import pallas as pl
from jax.experimental.pallas import tpu as pltpu
from jax.experimental.pallas import tpu_sc as plsc

EMB = 64
SCALE = math.sqrt(EMB)
NC = 2
NS = 16
NW = NC * NS
LANES = 16
CHUNK = 128
PITCH = 129
NBUF = 4
LA = 2


def _emb_body(n_chunks, nb, tok_hbm, tab_hbm, out_hbm,
              idx_v, a_v, t_v, sem_i, sem_g, sem_w):
    wid = lax.axis_index("s") * NC + lax.axis_index("c")
    c0 = wid * n_chunks

    def fire_idx(j):
        c = c0 + j
        s, b = c // nb, c % nb
        pltpu.async_copy(tok_hbm.at[s, pl.ds(b * CHUNK, CHUNK)],
                         idx_v.at[j % NBUF], sem_i)

    def fire_gather(j):
        pltpu.async_copy(tab_hbm.at[idx_v.at[j % NBUF]], a_v.at[j % NBUF], sem_g)

    def wait_idx():
        pltpu.make_async_copy(tok_hbm.at[0, pl.ds(0, CHUNK)],
                              idx_v.at[0], sem_i).wait()

    def wait_gather():
        pltpu.make_async_copy(tab_hbm.at[pl.ds(0, CHUNK)],
                              a_v.at[0], sem_g).wait()

    def wait_write():
        for _ in range(EMB // 8):
            pltpu.make_async_copy(t_v.at[0, pl.ds(0, 8), pl.ds(0, CHUNK)],
                                  out_hbm.at[0, 0, 0], sem_w).wait()

    lanes = lax.iota(jnp.int32, LANES)
    rowvecs = [lanes + c * LANES for c in range(EMB // LANES)]

    for j in range(min(NBUF, n_chunks)):
        fire_idx(j)
    for j in range(min(LA, n_chunks)):
        wait_idx()
        fire_gather(j)

    def step(j, _):
        slot = j % NBUF
        c = c0 + j
        s, b = c // nb, c % nb

        @pl.when(j >= LA)
        def _():
            wait_write()

        @pl.when(j + LA < n_chunks)
        def _():
            wait_idx()
            fire_gather(j + LA)

        wait_gather()

        @plsc.parallel_loop(0, CHUNK, unroll=8)
        def _(bi):
            cols = jnp.full((LANES,), bi, dtype=jnp.int32)
            for c4 in range(EMB // LANES):
                v = a_v[slot, bi, pl.ds(c4 * LANES, LANES)] * SCALE
                plsc.store_scatter(t_v.at[slot], [rowvecs[c4], cols], v)

        for g in range(EMB // 8):
            pltpu.async_copy(t_v.at[slot, pl.ds(g * 8, 8), pl.ds(0, CHUNK)],
                             out_hbm.at[s, g, b], sem_w)

        @pl.when(j + NBUF < n_chunks)
        def _():
            fire_idx(j + NBUF)
        return 0

    lax.fori_loop(0, n_chunks, step, 0)
    for _ in range(min(LA, n_chunks)):
        wait_write()


def kernel(tokens, table):
    bsz, s_len = tokens.shape
    nb = bsz // CHUNK
    total_chunks = s_len * nb
    assert total_chunks % NW == 0
    n_chunks = total_chunks // NW

    tok_t = tokens.T.astype(jnp.int32)
    t3 = lax.optimization_barrier(table.T).T

    mesh = plsc.VectorSubcoreMesh(
        core_axis_name="c", subcore_axis_name="s", num_cores=NC, num_subcores=NS
    )
    emb = pl.kernel(
        functools.partial(_emb_body, n_chunks, nb),
        out_type=jax.ShapeDtypeStruct((s_len, EMB // 8, nb, 8, CHUNK), jnp.float32),
        mesh=mesh,
        scratch_types=[
            pltpu.VMEM((NBUF, CHUNK), jnp.int32),
            pltpu.VMEM((NBUF, CHUNK, EMB), jnp.float32),
            pltpu.VMEM((NBUF, EMB, PITCH), jnp.float32),
            pltpu.SemaphoreType.DMA,
            pltpu.SemaphoreType.DMA,
            pltpu.SemaphoreType.DMA,
        ],
        compiler_params=pltpu.CompilerParams(
            use_tc_tiling_on_sc=False,
            needs_layout_passes=False,
            disable_bounds_checks=True,
        ),
    )
    out5 = emb(tok_t, t3)
    return out5.transpose(2, 4, 0, 1, 3).reshape(bsz, s_len, EMB)


# speedup vs baseline: 2.5956x; 1.1474x over previous
import functools
import math

import jax
import jax.numpy as jnp
from jax import lax
from jax.experimental import pallas as pl
from jax.experimental.pallas import tpu as pltpu
from jax.experimental.pallas import tpu_sc as plsc

EMB = 64
VOCAB = 1000000
SCALE = math.sqrt(EMB)
NC = 2
NS = 16
NW = NC * NS
LANES = 16
CHUNK = 128
PITCH = 129
NBUF = 4
LA = 2


def _emb_body(n_chunks, nb, tok_hbm, tab_hbm, out_hbm,
              idx_v, a_v, t_v, sem_i, sem_g, sem_w):
    wid = lax.axis_index("s") * NC + lax.axis_index("c")
    c0 = wid * n_chunks

    def fire_idx(j):
        c = c0 + j
        s, b = c // nb, c % nb
        pltpu.async_copy(tok_hbm.at[s, pl.ds(b * CHUNK, CHUNK)],
                         idx_v.at[j % NBUF], sem_i)

    def prep_gather(j):
        slot = j % NBUF
        half = VOCAB // 2

        @plsc.parallel_loop(0, CHUNK, LANES, unroll=4)
        def _(k):
            t = idx_v[slot, pl.ds(k, LANES)]
            idx_v[slot, pl.ds(k, LANES)] = jnp.where(
                t < half, t + t, t + t - (VOCAB - 1))

        pltpu.async_copy(tab_hbm.at[idx_v.at[slot]], a_v.at[slot], sem_g)

    def wait_idx():
        pltpu.make_async_copy(tok_hbm.at[0, pl.ds(0, CHUNK)],
                              idx_v.at[0], sem_i).wait()

    def wait_gather():
        pltpu.make_async_copy(tab_hbm.at[pl.ds(0, CHUNK)],
                              a_v.at[0], sem_g).wait()

    def wait_write():
        for _ in range(EMB // 8):
            pltpu.make_async_copy(t_v.at[0, pl.ds(0, 8), pl.ds(0, CHUNK)],
                                  out_hbm.at[0, 0, 0], sem_w).wait()

    lanes = lax.iota(jnp.int32, LANES)
    rowvecs = [lanes + c * LANES for c in range(EMB // LANES)]

    for j in range(min(NBUF, n_chunks)):
        fire_idx(j)
    for j in range(min(LA, n_chunks)):
        wait_idx()
        prep_gather(j)

    def step(j, _):
        slot = j % NBUF
        c = c0 + j
        s, b = c // nb, c % nb

        @pl.when(j >= LA)
        def _():
            wait_write()

        @pl.when(j + LA < n_chunks)
        def _():
            wait_idx()
            prep_gather(j + LA)

        wait_gather()

        @plsc.parallel_loop(0, CHUNK, unroll=8)
        def _(bi):
            cols = jnp.full((LANES,), bi, dtype=jnp.int32)
            for c4 in range(EMB // LANES):
                v = a_v[slot, bi, pl.ds(c4 * LANES, LANES)] * SCALE
                plsc.store_scatter(t_v.at[slot], [rowvecs[c4], cols], v)

        for g in range(EMB // 8):
            pltpu.async_copy(t_v.at[slot, pl.ds(g * 8, 8), pl.ds(0, CHUNK)],
                             out_hbm.at[s, g, b], sem_w)

        @pl.when(j + NBUF < n_chunks)
        def _():
            fire_idx(j + NBUF)
        return 0

    lax.fori_loop(0, n_chunks, step, 0)
    for _ in range(min(LA, n_chunks)):
        wait_write()


def kernel(tokens, table):
    bsz, s_len = tokens.shape
    nb = bsz // CHUNK
    total_chunks = s_len * nb
    assert total_chunks % NW == 0
    n_chunks = total_chunks // NW

    tok_t = tokens.T.astype(jnp.int32)
    half = table.shape[0] // 2
    t2 = jnp.concatenate([table[:half], table[half:]], axis=1)
    t3 = t2.reshape(table.shape[0], EMB)

    mesh = plsc.VectorSubcoreMesh(
        core_axis_name="c", subcore_axis_name="s", num_cores=NC, num_subcores=NS
    )
    emb = pl.kernel(
        functools.partial(_emb_body, n_chunks, nb),
        out_type=jax.ShapeDtypeStruct((s_len, EMB // 8, nb, 8, CHUNK), jnp.float32),
        mesh=mesh,
        scratch_types=[
            pltpu.VMEM((NBUF, CHUNK), jnp.int32),
            pltpu.VMEM((NBUF, CHUNK, EMB), jnp.float32),
            pltpu.VMEM((NBUF, EMB, PITCH), jnp.float32),
            pltpu.SemaphoreType.DMA,
            pltpu.SemaphoreType.DMA,
            pltpu.SemaphoreType.DMA,
        ],
        compiler_params=pltpu.CompilerParams(
            use_tc_tiling_on_sc=False,
            needs_layout_passes=False,
            disable_bounds_checks=True,
        ),
    )
    out5 = emb(tok_t, t3)
    return out5.transpose(2, 4, 0, 1, 3).reshape(bsz, s_len, EMB)
